# Initial kernel scaffold; baseline (speedup 1.0000x reference)
import jax
import jax.numpy as jnp
from jax.experimental import pallas as pl


def kernel(embeddings, edge_index, adj_values, W):
    raise NotImplementedError("write your pallas kernel here")


# speedup vs baseline: 4.4745x; 4.4745x over previous
import functools

import jax
import jax.numpy as jnp
from jax import lax
from jax.experimental import pallas as pl
from jax.experimental.pallas import tpu as pltpu
from jax.experimental.pallas import tpu_sc as plsc

N_NODES = 10000
N_EDGES = 320000
D = 128

NC = 2
NS = 16
NW = NC * NS

E_PER_TILE = N_EDGES // NW
BLK = 80
NBLK = E_PER_TILE // BLK
ROWS_PER_TILE = N_NODES // NS


def _spmm_body(dst_hbm, src_hbm, vals_hbm, emb_hbm, out_hbm,
               idx_v, dst_v, vals_v, rows_v, acc_sh, sem):
    c = lax.axis_index("c")
    s = lax.axis_index("s")

    zero16 = jnp.zeros((16,), jnp.float32)

    def zrow(r, carry):
        for cc in range(8):
            rows_v[r, pl.ds(cc * 16, 16)] = zero16
        return carry

    lax.fori_loop(0, BLK, zrow, 0)

    def zero_chunk(k, carry):
        @pl.when(lax.rem(k, NS) == s)
        def _():
            off = pl.multiple_of(k * BLK, 8)
            pltpu.sync_copy(rows_v, acc_sh.at[pl.ds(off, BLK)])
        return carry

    lax.fori_loop(0, N_NODES // BLK, zero_chunk, 0)
    plsc.subcore_barrier()

    ebase = (c * NS + s) * E_PER_TILE

    def blk_body(b, carry):
        off = ebase + b * BLK
        pltpu.sync_copy(src_hbm.at[pl.ds(off, BLK)], idx_v)
        pltpu.sync_copy(dst_hbm.at[pl.ds(off, BLK)], dst_v)
        pltpu.sync_copy(vals_hbm.at[pl.ds(off, BLK)], vals_v)
        pltpu.async_copy(emb_hbm.at[idx_v], rows_v, sem).wait()

        def scale(g, inner):
            vvec = vals_v[pl.ds(g * 16, 16)]
            for j in range(16):
                vb = jnp.full((16,), vvec[j], jnp.float32)
                e = g * 16 + j
                for cc in range(8):
                    rows_v[e, pl.ds(cc * 16, 16)] = (
                        rows_v[e, pl.ds(cc * 16, 16)] * vb)
            return inner

        lax.fori_loop(0, BLK // 16, scale, 0)
        pltpu.sync_copy(rows_v, acc_sh.at[dst_v], add=True)
        return carry

    lax.fori_loop(0, NBLK, blk_body, 0)
    plsc.subcore_barrier()

    def drain_chunk(k, carry):
        @pl.when(lax.rem(k, NS) == s)
        def _():
            off = pl.multiple_of(k * BLK, 8)
            pltpu.sync_copy(acc_sh.at[pl.ds(off, BLK)],
                            out_hbm.at[c, pl.ds(off, BLK)])
        return carry

    lax.fori_loop(0, N_NODES // BLK, drain_chunk, 0)


_spmm = functools.partial(
    pl.kernel,
    out_type=jax.ShapeDtypeStruct((NC, N_NODES, D), jnp.float32),
    mesh=plsc.VectorSubcoreMesh(core_axis_name="c", subcore_axis_name="s"),
    scratch_types=[
        pltpu.VMEM((BLK,), jnp.int32),
        pltpu.VMEM((BLK,), jnp.int32),
        pltpu.VMEM((BLK,), jnp.float32),
        pltpu.VMEM((BLK, D), jnp.float32),
        pltpu.VMEM_SHARED((N_NODES, D), jnp.float32),
        pltpu.SemaphoreType.DMA,
    ],
)(_spmm_body)


BM = 400


def _mm_body(p_ref, w_ref, o_ref):
    p = p_ref[0] + p_ref[1]
    o_ref[...] = jnp.dot(p, w_ref[...], preferred_element_type=jnp.float32)


def _matmul(partials, W):
    return pl.pallas_call(
        _mm_body,
        grid=(N_NODES // BM,),
        in_specs=[
            pl.BlockSpec((NC, BM, D), lambda i: (0, i, 0)),
            pl.BlockSpec((D, D), lambda i: (0, 0)),
        ],
        out_specs=pl.BlockSpec((BM, D), lambda i: (i, 0)),
        out_shape=jax.ShapeDtypeStruct((N_NODES, D), jnp.float32),
    )(partials, W)


def kernel(embeddings, edge_index, adj_values, W):
    dst = edge_index[0]
    src = edge_index[1]
    partials = _spmm(dst, src, adj_values, embeddings)
    return _matmul(partials, W)


# speedup vs baseline: 10.9377x; 2.4445x over previous
import functools

import jax
import jax.numpy as jnp
from jax import lax
from jax.experimental import pallas as pl
from jax.experimental.pallas import tpu as pltpu
from jax.experimental.pallas import tpu_sc as plsc

N_NODES = 10000
N_EDGES = 320000
D = 128

NC = 2
NS = 16
NW = NC * NS

E_PER_TILE = N_EDGES // NW
BLK = 80
NBLK = E_PER_TILE // BLK
CHUNK = 80
NCHUNK = N_NODES // CHUNK


def _spmm_body(dst_hbm, src_hbm, vals_hbm, emb_hbm, out_hbm,
               sbuf, dbuf, vbuf, rows, acc_sh, esem, gsem):
    c = lax.axis_index("c")
    s = lax.axis_index("s")
    ebase = (c * NS + s) * E_PER_TILE

    zero16 = jnp.zeros((16,), jnp.float32)

    def zrow(r, carry):
        for cc in range(8):
            rows[0][r, pl.ds(cc * 16, 16)] = zero16
        return carry

    lax.fori_loop(0, CHUNK, zrow, 0)

    def zero_chunk(k, carry):
        @pl.when(lax.rem(k, NS) == s)
        def _():
            off = pl.multiple_of(k * CHUNK, 8)
            pltpu.sync_copy(rows[0], acc_sh.at[pl.ds(off, CHUNK)])
        return carry

    lax.fori_loop(0, NCHUNK, zero_chunk, 0)
    plsc.subcore_barrier()

    def eload(b, r):
        off = ebase + b * BLK
        yield pltpu.make_async_copy(src_hbm.at[pl.ds(off, BLK)], sbuf[r], esem[r])
        yield pltpu.make_async_copy(dst_hbm.at[pl.ds(off, BLK)], dbuf[r], esem[r])
        yield pltpu.make_async_copy(vals_hbm.at[pl.ds(off, BLK)], vbuf[r], esem[r])

    def eload_start(b, r):
        for cp in eload(b, r):
            cp.start()

    def eload_wait(b, r):
        for cp in eload(b, r):
            cp.wait()

    def gather_start(r, q):
        pltpu.make_async_copy(emb_hbm.at[sbuf[r]], rows[q], gsem[q]).start()

    def gather_wait(r, q):
        pltpu.make_async_copy(emb_hbm.at[sbuf[r]], rows[q], gsem[q]).wait()

    def process(r, q):
        gather_wait(r, q)
        buf = rows[q]
        vals = vbuf[r]

        def scale(g, carry):
            vvec = vals[pl.ds(g * 16, 16)]
            for j in range(16):
                vb = jnp.full((16,), vvec[j], jnp.float32)
                e = g * 16 + j
                for cc in range(8):
                    buf[e, pl.ds(cc * 16, 16)] = buf[e, pl.ds(cc * 16, 16)] * vb
            return carry

        lax.fori_loop(0, BLK // 16, scale, 0)
        pltpu.sync_copy(buf, acc_sh.at[dbuf[r]], add=True)

    eload_start(0, 0)
    eload_start(1, 1)
    eload_start(2, 2)
    eload_wait(0, 0)
    gather_start(0, 0)
    eload_wait(1, 1)
    gather_start(1, 1)

    def quad(i, carry):
        for k in range(4):
            b = 4 * i + k
            nring = (k + 3) % 4
            if k >= 2:
                @pl.when(b + 3 < NBLK)
                def _():
                    eload_start(b + 3, nring)
            else:
                eload_start(b + 3, nring)
            process(k, k % 2)
            if k == 3:
                @pl.when(b + 2 < NBLK)
                def _():
                    eload_wait(b + 2, (k + 2) % 4)
                    gather_start((k + 2) % 4, k % 2)
            else:
                eload_wait(b + 2, (k + 2) % 4)
                gather_start((k + 2) % 4, k % 2)
        return carry

    lax.fori_loop(0, NBLK // 4, quad, 0)
    process(0, 0)
    plsc.subcore_barrier()

    def drain_chunk(k, carry):
        @pl.when(lax.rem(k, NS) == s)
        def _():
            off = pl.multiple_of(k * CHUNK, 8)
            pltpu.sync_copy(acc_sh.at[pl.ds(off, CHUNK)],
                            out_hbm.at[c, pl.ds(off, CHUNK)])
        return carry

    lax.fori_loop(0, NCHUNK, drain_chunk, 0)


_spmm = functools.partial(
    pl.kernel,
    out_type=jax.ShapeDtypeStruct((NC, N_NODES, D), jnp.float32),
    mesh=plsc.VectorSubcoreMesh(core_axis_name="c", subcore_axis_name="s"),
    scratch_types=[
        [pltpu.VMEM((BLK,), jnp.int32) for _ in range(4)],
        [pltpu.VMEM((BLK,), jnp.int32) for _ in range(4)],
        [pltpu.VMEM((BLK,), jnp.float32) for _ in range(4)],
        [pltpu.VMEM((BLK, D), jnp.float32) for _ in range(2)],
        pltpu.VMEM_SHARED((N_NODES, D), jnp.float32),
        [pltpu.SemaphoreType.DMA for _ in range(4)],
        [pltpu.SemaphoreType.DMA for _ in range(2)],
    ],
)(_spmm_body)


BM = 400


def _mm_body(p_ref, w_ref, o_ref):
    p = p_ref[0] + p_ref[1]
    o_ref[...] = jnp.dot(p, w_ref[...], preferred_element_type=jnp.float32)


def _matmul(partials, W):
    return pl.pallas_call(
        _mm_body,
        grid=(N_NODES // BM,),
        in_specs=[
            pl.BlockSpec((NC, BM, D), lambda i: (0, i, 0)),
            pl.BlockSpec((D, D), lambda i: (0, 0)),
        ],
        out_specs=pl.BlockSpec((BM, D), lambda i: (i, 0)),
        out_shape=jax.ShapeDtypeStruct((N_NODES, D), jnp.float32),
    )(partials, W)


def kernel(embeddings, edge_index, adj_values, W):
    dst = edge_index[0]
    src = edge_index[1]
    partials = _spmm(dst, src, adj_values, embeddings)
    return _matmul(partials, W)
